# baseline (speedup 1.0000x reference)
import functools

import jax
import jax.numpy as jnp
from jax import lax
from jax.experimental import pallas as pl
from jax.experimental.pallas import tpu as pltpu
from jax.experimental.pallas import tpu_sc as plsc

N = 10000
NP = 10240
E = 320000
NC, NS = 2, 16
NW = NC * NS
CHUNK = 128
K = 80
KI = 16
EP = NW * K * CHUNK
STRIPE = NP // NS
KIA = 8
K0 = 120
K1 = 40

_MESH = plsc.VectorSubcoreMesh(core_axis_name="c", subcore_axis_name="s")


def _make_sc_agg():
    out_type = jax.ShapeDtypeStruct((NC, NP, 128), jnp.float32)
    scratch = [
        pltpu.VMEM((KIA, CHUNK), jnp.int32),
        pltpu.VMEM((KIA, CHUNK), jnp.int32),
        pltpu.VMEM((CHUNK, 128), jnp.float32),
        pltpu.VMEM((CHUNK, 128), jnp.float32),
        pltpu.VMEM_SHARED((NP, 128), jnp.float32),
        pltpu.SemaphoreType.DMA,
        pltpu.SemaphoreType.DMA,
    ]

    def body(m_hbm, srcw, dstw, z_hbm, out_hbm, sidx, didx, rows_a, rows_b,
             acc_sh, sga, sgb):
        c = lax.axis_index("c")
        s = lax.axis_index("s")
        wid = c * NS + s
        sl = pl.ds(s * STRIPE, STRIPE)

        def wait_g(sem, dst):
            pltpu.make_async_copy(m_hbm.at[pl.ds(0, CHUNK)], dst, sem).wait()

        pltpu.sync_copy(z_hbm.at[sl], acc_sh.at[sl])
        plsc.subcore_barrier()

        def blk(bi, carry):
            pltpu.sync_copy(srcw.at[wid, pl.ds(bi * KIA, KIA)], sidx)
            pltpu.sync_copy(dstw.at[wid, pl.ds(bi * KIA, KIA)], didx)
            pltpu.async_copy(m_hbm.at[sidx.at[0]], rows_a, sga)
            pltpu.async_copy(m_hbm.at[sidx.at[1]], rows_b, sgb)

            def pair(t, c2):
                j0 = 2 * t
                wait_g(sga, rows_a)
                pltpu.sync_copy(rows_a, acc_sh.at[didx.at[j0]], add=True)
                pltpu.async_copy(m_hbm.at[sidx.at[j0 + 2]], rows_a, sga)
                wait_g(sgb, rows_b)
                pltpu.sync_copy(rows_b, acc_sh.at[didx.at[j0 + 1]], add=True)
                pltpu.async_copy(m_hbm.at[sidx.at[j0 + 3]], rows_b, sgb)
                return c2

            lax.fori_loop(0, KIA // 2 - 1, pair, 0)
            wait_g(sga, rows_a)
            pltpu.sync_copy(rows_a, acc_sh.at[didx.at[KIA - 2]], add=True)
            wait_g(sgb, rows_b)
            pltpu.sync_copy(rows_b, acc_sh.at[didx.at[KIA - 1]], add=True)
            return carry

        nblocks = jnp.where(c == 0, K0 // KIA, K1 // KIA)
        lax.fori_loop(0, nblocks, blk, 0)
        plsc.subcore_barrier()

        pltpu.sync_copy(acc_sh.at[sl], out_hbm.at[c, sl])

    return pl.kernel(body, out_type=out_type, mesh=_MESH,
                     scratch_types=scratch)


def _make_sc_deg():
    out_type = jax.ShapeDtypeStruct((NC, NP, 128), jnp.float32)
    scratch = [
        pltpu.VMEM((KI, CHUNK), jnp.int32),
        pltpu.VMEM((CHUNK, 128), jnp.float32),
        pltpu.VMEM_SHARED((NP, 128), jnp.float32),
    ]

    def body(dstw, z_hbm, ones_hbm, out_hbm, didx, ones_v, acc_sh):
        c = lax.axis_index("c")
        s = lax.axis_index("s")
        wid = c * NS + s
        sl = pl.ds(s * STRIPE, STRIPE)

        pltpu.sync_copy(z_hbm.at[sl], acc_sh.at[sl])
        pltpu.sync_copy(ones_hbm, ones_v)
        plsc.subcore_barrier()

        def blk(bi, carry):
            pltpu.sync_copy(dstw.at[wid, pl.ds(bi * KI, KI)], didx)

            def chunk(j, c2):
                pltpu.sync_copy(ones_v, acc_sh.at[didx.at[j]], add=True)
                return c2

            lax.fori_loop(0, KI, chunk, 0)
            return carry

        lax.fori_loop(0, K // KI, blk, 0)
        plsc.subcore_barrier()

        pltpu.sync_copy(acc_sh.at[sl], out_hbm.at[c, sl])

    return pl.kernel(body, out_type=out_type, mesh=_MESH,
                     scratch_types=scratch)


_AGG128 = _make_sc_agg()
_DEG = _make_sc_deg()

BR = 1024


def _tc0_body(x_ref, wn_ref, ws_ref, b_ref, m_ref, s_ref):
    xb = x_ref[...]
    m_ref[...] = jnp.dot(xb, wn_ref[...], preferred_element_type=jnp.float32)
    s_ref[...] = (jnp.dot(xb, ws_ref[...], preferred_element_type=jnp.float32)
                  + b_ref[...])


def _tc0(x, wn, ws, b):
    return pl.pallas_call(
        _tc0_body,
        grid=(NP // BR,),
        in_specs=[
            pl.BlockSpec((BR, 128), lambda r: (r, 0)),
            pl.BlockSpec((128, 128), lambda r: (0, 0)),
            pl.BlockSpec((128, 128), lambda r: (0, 0)),
            pl.BlockSpec((1, 128), lambda r: (0, 0)),
        ],
        out_specs=[pl.BlockSpec((BR, 128), lambda r: (r, 0)),
                   pl.BlockSpec((BR, 128), lambda r: (r, 0))],
        out_shape=[jax.ShapeDtypeStruct((NP, 128), jnp.float32)] * 2,
    )(x, wn, ws, b)


def _combine(s_ref, p_ref, degp_ref):
    deg = degp_ref[0, :, 0:1] + degp_ref[1, :, 0:1]
    invd = 1.0 / jnp.maximum(deg, 1.0)
    return s_ref[...] + (p_ref[0] + p_ref[1]) * invd


def _make_tc_mid(DN, DS):
    def tc_mid_body(s_ref, p_ref, degp_ref, wn_ref, ws_ref, b_ref,
                    m_ref, so_ref):
        h = jnp.maximum(_combine(s_ref, p_ref, degp_ref), 0.0)
        m_ref[...] = jnp.dot(h, wn_ref[...],
                             preferred_element_type=jnp.float32)
        so_ref[...] = (jnp.dot(h, ws_ref[...],
                               preferred_element_type=jnp.float32)
                       + b_ref[...])

    def tc_mid(s, p, degp, wn, ws, b):
        return pl.pallas_call(
            tc_mid_body,
            grid=(NP // BR,),
            in_specs=[
                pl.BlockSpec((BR, 128), lambda r: (r, 0)),
                pl.BlockSpec((NC, BR, 128), lambda r: (0, r, 0)),
                pl.BlockSpec((NC, BR, 128), lambda r: (0, r, 0)),
                pl.BlockSpec((128, DN), lambda r: (0, 0)),
                pl.BlockSpec((128, DS), lambda r: (0, 0)),
                pl.BlockSpec((1, DS), lambda r: (0, 0)),
            ],
            out_specs=[pl.BlockSpec((BR, DN), lambda r: (r, 0)),
                       pl.BlockSpec((BR, DS), lambda r: (r, 0))],
            out_shape=[jax.ShapeDtypeStruct((NP, DN), jnp.float32),
                       jax.ShapeDtypeStruct((NP, DS), jnp.float32)],
        )(s, p, degp, wn, ws, b)

    return tc_mid


_TC_MID128 = _make_tc_mid(128, 128)
_TC_MID2 = _make_tc_mid(128, 64)


def _tc_fin_body(s_ref, p_ref, degp_ref, o_ref):
    deg = degp_ref[0, :, 0:1] + degp_ref[1, :, 0:1]
    invd = 1.0 / jnp.maximum(deg, 1.0)
    o_ref[...] = s_ref[...] + (p_ref[0, :, :64] + p_ref[1, :, :64]) * invd


def _tc_fin(s, p, degp):
    return pl.pallas_call(
        _tc_fin_body,
        grid=(NP // BR,),
        in_specs=[
            pl.BlockSpec((BR, 64), lambda r: (r, 0)),
            pl.BlockSpec((NC, BR, 128), lambda r: (0, r, 0)),
            pl.BlockSpec((NC, BR, 128), lambda r: (0, r, 0)),
        ],
        out_specs=pl.BlockSpec((BR, 64), lambda r: (r, 0)),
        out_shape=jax.ShapeDtypeStruct((NP, 64), jnp.float32),
    )(s, p, degp)


@jax.jit
def kernel(x, edge_index, W_self0, W_neigh0, b0, W_self1, W_neigh1, b1,
           W_self2, W_neigh2, b2):
    xp = jnp.pad(x, ((0, NP - N), (0, 0)))
    src_flat = jnp.pad(edge_index[0], (0, EP - E))
    dst_flat = jnp.pad(edge_index[1], (0, EP - E), constant_values=N)
    dst = dst_flat.reshape(NW, K, CHUNK)

    def weighted(e_flat):
        kmax = max(K0, K1)
        n0 = NS * K0 * CHUNK
        e0 = e_flat[:n0].reshape(NS, K0, CHUNK)
        e0 = jnp.pad(e0, ((0, 0), (0, kmax - K0), (0, 0)))
        e1 = e_flat[n0:].reshape(NS, K1, CHUNK)
        e1 = jnp.pad(e1, ((0, 0), (0, kmax - K1), (0, 0)))
        return jnp.concatenate([e0, e1], axis=0)

    src_w = weighted(src_flat)
    dst_w = weighted(dst_flat)
    z128 = jnp.zeros((NP, 128), jnp.float32)
    ones = jnp.ones((CHUNK, 128), jnp.float32)
    wn2p = jnp.pad(W_neigh2, ((0, 0), (0, 128 - 64)))

    degp = _DEG(dst, z128, ones)
    m0, s0 = _tc0(xp, W_neigh0, W_self0, b0.reshape(1, -1))
    p0 = _AGG128(m0, src_w, dst_w, z128)
    m1, s1 = _TC_MID128(s0, p0, degp, W_neigh1, W_self1, b1.reshape(1, -1))
    p1 = _AGG128(m1, src_w, dst_w, z128)
    m2, s2 = _TC_MID2(s1, p1, degp, wn2p, W_self2, b2.reshape(1, -1))
    p2 = _AGG128(m2, src_w, dst_w, z128)
    out = _tc_fin(s2, p2, degp)
    return out[:N]


# speedup vs baseline: 1.3816x; 1.3816x over previous
import functools

import jax
import jax.numpy as jnp
from jax import lax
from jax.experimental import pallas as pl
from jax.experimental.pallas import tpu as pltpu
from jax.experimental.pallas import tpu_sc as plsc

N = 10000
NP = 10240
E = 320000
NC, NS = 2, 16
NW = NC * NS
CHUNK = 128
K = 80
KI = 40
KID = 16
EP = NW * K * CHUNK
STRIPE = NP // NS

_MESH = plsc.VectorSubcoreMesh(core_axis_name="c", subcore_axis_name="s")


def _make_sc_agg():
    out_type = jax.ShapeDtypeStruct((NC, NP, 128), jnp.float32)
    scratch = [
        pltpu.VMEM((KI, CHUNK), jnp.int32),
        pltpu.VMEM((KI, CHUNK), jnp.int32),
        pltpu.VMEM((CHUNK, 128), jnp.float32),
        pltpu.VMEM((CHUNK, 128), jnp.float32),
        pltpu.VMEM_SHARED((NP, 128), jnp.float32),
        pltpu.SemaphoreType.DMA,
        pltpu.SemaphoreType.DMA,
    ]

    def body(m_hbm, srcw, dstw, z_hbm, out_hbm, sidx, didx, rows_a, rows_b,
             acc_sh, sga, sgb):
        c = lax.axis_index("c")
        s = lax.axis_index("s")
        wid = c * NS + s
        sl = pl.ds(s * STRIPE, STRIPE)

        def wait_g(sem, dst):
            pltpu.make_async_copy(m_hbm.at[pl.ds(0, CHUNK)], dst, sem).wait()

        pltpu.sync_copy(z_hbm.at[sl], acc_sh.at[sl])
        plsc.subcore_barrier()

        def blk(bi, carry):
            pltpu.sync_copy(srcw.at[wid, pl.ds(bi * KI, KI)], sidx)
            pltpu.sync_copy(dstw.at[wid, pl.ds(bi * KI, KI)], didx)
            pltpu.async_copy(m_hbm.at[sidx.at[0]], rows_a, sga)
            pltpu.async_copy(m_hbm.at[sidx.at[1]], rows_b, sgb)

            def pair(t, c2):
                j0 = 2 * t
                wait_g(sga, rows_a)
                pltpu.sync_copy(rows_a, acc_sh.at[didx.at[j0]], add=True)
                pltpu.async_copy(m_hbm.at[sidx.at[j0 + 2]], rows_a, sga)
                wait_g(sgb, rows_b)
                pltpu.sync_copy(rows_b, acc_sh.at[didx.at[j0 + 1]], add=True)
                pltpu.async_copy(m_hbm.at[sidx.at[j0 + 3]], rows_b, sgb)
                return c2

            lax.fori_loop(0, KI // 2 - 1, pair, 0)
            wait_g(sga, rows_a)
            pltpu.sync_copy(rows_a, acc_sh.at[didx.at[KI - 2]], add=True)
            wait_g(sgb, rows_b)
            pltpu.sync_copy(rows_b, acc_sh.at[didx.at[KI - 1]], add=True)
            return carry

        lax.fori_loop(0, K // KI, blk, 0)
        plsc.subcore_barrier()

        pltpu.sync_copy(acc_sh.at[sl], out_hbm.at[c, sl])

    return pl.kernel(body, out_type=out_type, mesh=_MESH,
                     scratch_types=scratch)


def _make_sc_deg():
    out_type = jax.ShapeDtypeStruct((NC, NP, 128), jnp.float32)
    scratch = [
        pltpu.VMEM((KID, CHUNK), jnp.int32),
        pltpu.VMEM((CHUNK, 128), jnp.float32),
        pltpu.VMEM_SHARED((NP, 128), jnp.float32),
    ]

    def body(dstw, z_hbm, ones_hbm, out_hbm, didx, ones_v, acc_sh):
        c = lax.axis_index("c")
        s = lax.axis_index("s")
        wid = c * NS + s
        sl = pl.ds(s * STRIPE, STRIPE)

        pltpu.sync_copy(z_hbm.at[sl], acc_sh.at[sl])
        pltpu.sync_copy(ones_hbm, ones_v)
        plsc.subcore_barrier()

        def blk(bi, carry):
            pltpu.sync_copy(dstw.at[wid, pl.ds(bi * KID, KID)], didx)

            def chunk(j, c2):
                pltpu.sync_copy(ones_v, acc_sh.at[didx.at[j]], add=True)
                return c2

            lax.fori_loop(0, KID, chunk, 0)
            return carry

        lax.fori_loop(0, K // KID, blk, 0)
        plsc.subcore_barrier()

        pltpu.sync_copy(acc_sh.at[sl], out_hbm.at[c, sl])

    return pl.kernel(body, out_type=out_type, mesh=_MESH,
                     scratch_types=scratch)


_AGG128 = _make_sc_agg()
_DEG = _make_sc_deg()

BR = 1024


def _tc0_body(x_ref, wn_ref, ws_ref, b_ref, m_ref, s_ref):
    xb = x_ref[...]
    m_ref[...] = jnp.dot(xb, wn_ref[...], preferred_element_type=jnp.float32)
    s_ref[...] = (jnp.dot(xb, ws_ref[...], preferred_element_type=jnp.float32)
                  + b_ref[...])


def _tc0(x, wn, ws, b):
    return pl.pallas_call(
        _tc0_body,
        grid=(NP // BR,),
        in_specs=[
            pl.BlockSpec((BR, 128), lambda r: (r, 0)),
            pl.BlockSpec((128, 128), lambda r: (0, 0)),
            pl.BlockSpec((128, 128), lambda r: (0, 0)),
            pl.BlockSpec((1, 128), lambda r: (0, 0)),
        ],
        out_specs=[pl.BlockSpec((BR, 128), lambda r: (r, 0)),
                   pl.BlockSpec((BR, 128), lambda r: (r, 0))],
        out_shape=[jax.ShapeDtypeStruct((NP, 128), jnp.float32)] * 2,
    )(x, wn, ws, b)


def _combine(s_ref, p_ref, degp_ref):
    deg = degp_ref[0, :, 0:1] + degp_ref[1, :, 0:1]
    invd = 1.0 / jnp.maximum(deg, 1.0)
    return s_ref[...] + (p_ref[0] + p_ref[1]) * invd


def _make_tc_mid(DN, DS):
    def tc_mid_body(s_ref, p_ref, degp_ref, wn_ref, ws_ref, b_ref,
                    m_ref, so_ref):
        h = jnp.maximum(_combine(s_ref, p_ref, degp_ref), 0.0)
        m_ref[...] = jnp.dot(h, wn_ref[...],
                             preferred_element_type=jnp.float32)
        so_ref[...] = (jnp.dot(h, ws_ref[...],
                               preferred_element_type=jnp.float32)
                       + b_ref[...])

    def tc_mid(s, p, degp, wn, ws, b):
        return pl.pallas_call(
            tc_mid_body,
            grid=(NP // BR,),
            in_specs=[
                pl.BlockSpec((BR, 128), lambda r: (r, 0)),
                pl.BlockSpec((NC, BR, 128), lambda r: (0, r, 0)),
                pl.BlockSpec((NC, BR, 128), lambda r: (0, r, 0)),
                pl.BlockSpec((128, DN), lambda r: (0, 0)),
                pl.BlockSpec((128, DS), lambda r: (0, 0)),
                pl.BlockSpec((1, DS), lambda r: (0, 0)),
            ],
            out_specs=[pl.BlockSpec((BR, DN), lambda r: (r, 0)),
                       pl.BlockSpec((BR, DS), lambda r: (r, 0))],
            out_shape=[jax.ShapeDtypeStruct((NP, DN), jnp.float32),
                       jax.ShapeDtypeStruct((NP, DS), jnp.float32)],
        )(s, p, degp, wn, ws, b)

    return tc_mid


_TC_MID128 = _make_tc_mid(128, 128)
_TC_MID2 = _make_tc_mid(128, 64)


def _tc_fin_body(s_ref, p_ref, degp_ref, o_ref):
    deg = degp_ref[0, :, 0:1] + degp_ref[1, :, 0:1]
    invd = 1.0 / jnp.maximum(deg, 1.0)
    o_ref[...] = s_ref[...] + (p_ref[0, :, :64] + p_ref[1, :, :64]) * invd


def _tc_fin(s, p, degp):
    return pl.pallas_call(
        _tc_fin_body,
        grid=(NP // BR,),
        in_specs=[
            pl.BlockSpec((BR, 64), lambda r: (r, 0)),
            pl.BlockSpec((NC, BR, 128), lambda r: (0, r, 0)),
            pl.BlockSpec((NC, BR, 128), lambda r: (0, r, 0)),
        ],
        out_specs=pl.BlockSpec((BR, 64), lambda r: (r, 0)),
        out_shape=jax.ShapeDtypeStruct((NP, 64), jnp.float32),
    )(s, p, degp)


@jax.jit
def kernel(x, edge_index, W_self0, W_neigh0, b0, W_self1, W_neigh1, b1,
           W_self2, W_neigh2, b2):
    xp = jnp.pad(x, ((0, NP - N), (0, 0)))
    src = jnp.pad(edge_index[0], (0, EP - E)).reshape(NW, K, CHUNK)
    dst = jnp.pad(edge_index[1], (0, EP - E),
                  constant_values=N).reshape(NW, K, CHUNK)
    z128 = jnp.zeros((NP, 128), jnp.float32)
    ones = jnp.ones((CHUNK, 128), jnp.float32)
    wn2p = jnp.pad(W_neigh2, ((0, 0), (0, 128 - 64)))

    degp = _DEG(dst, z128, ones)
    m0, s0 = _tc0(xp, W_neigh0, W_self0, b0.reshape(1, -1))
    p0 = _AGG128(m0, src, dst, z128)
    m1, s1 = _TC_MID128(s0, p0, degp, W_neigh1, W_self1, b1.reshape(1, -1))
    p1 = _AGG128(m1, src, dst, z128)
    m2, s2 = _TC_MID2(s1, p1, degp, wn2p, W_self2, b2.reshape(1, -1))
    p2 = _AGG128(m2, src, dst, z128)
    out = _tc_fin(s2, p2, degp)
    return out[:N]


# speedup vs baseline: 1.3825x; 1.0007x over previous
import jax
import jax.numpy as jnp
from jax import lax
from jax.experimental import pallas as pl
from jax.experimental.pallas import tpu as pltpu
from jax.experimental.pallas import tpu_sc as plsc

N = 10000
NP = 10240
E = 320000
NC, NS = 2, 16
NW = NC * NS
CHUNK = 128
K = 80
KI = 40
KID = 16
EP = NW * K * CHUNK
STRIPE = NP // NS

_MESH = plsc.VectorSubcoreMesh(core_axis_name="c", subcore_axis_name="s")


def _make_sc_agg():
    out_type = jax.ShapeDtypeStruct((NC, NP, 128), jnp.float32)
    scratch = [
        pltpu.VMEM((KI, CHUNK), jnp.int32),
        pltpu.VMEM((KI, CHUNK), jnp.int32),
        pltpu.VMEM((CHUNK, 128), jnp.float32),
        pltpu.VMEM((CHUNK, 128), jnp.float32),
        pltpu.VMEM_SHARED((NP, 128), jnp.float32),
        pltpu.SemaphoreType.DMA,
        pltpu.SemaphoreType.DMA,
    ]

    def body(m_hbm, srcw, dstw, z_hbm, out_hbm, sidx, didx, rows_a, rows_b,
             acc_sh, sga, sgb):
        c = lax.axis_index("c")
        s = lax.axis_index("s")
        wid = c * NS + s
        sl = pl.ds(s * STRIPE, STRIPE)

        def wait_g(sem, dst):
            pltpu.make_async_copy(m_hbm.at[pl.ds(0, CHUNK)], dst, sem).wait()

        pltpu.sync_copy(z_hbm.at[sl], acc_sh.at[sl])
        plsc.subcore_barrier()

        def blk(bi, carry):
            pltpu.sync_copy(srcw.at[wid, pl.ds(bi * KI, KI)], sidx)
            pltpu.sync_copy(dstw.at[wid, pl.ds(bi * KI, KI)], didx)
            pltpu.async_copy(m_hbm.at[sidx.at[0]], rows_a, sga)
            pltpu.async_copy(m_hbm.at[sidx.at[1]], rows_b, sgb)

            def pair(t, c2):
                j0 = 2 * t
                wait_g(sga, rows_a)
                pltpu.sync_copy(rows_a, acc_sh.at[didx.at[j0]], add=True)
                pltpu.async_copy(m_hbm.at[sidx.at[j0 + 2]], rows_a, sga)
                wait_g(sgb, rows_b)
                pltpu.sync_copy(rows_b, acc_sh.at[didx.at[j0 + 1]], add=True)
                pltpu.async_copy(m_hbm.at[sidx.at[j0 + 3]], rows_b, sgb)
                return c2

            lax.fori_loop(0, KI // 2 - 1, pair, 0)
            wait_g(sga, rows_a)
            pltpu.sync_copy(rows_a, acc_sh.at[didx.at[KI - 2]], add=True)
            wait_g(sgb, rows_b)
            pltpu.sync_copy(rows_b, acc_sh.at[didx.at[KI - 1]], add=True)
            return carry

        lax.fori_loop(0, K // KI, blk, 0)
        plsc.subcore_barrier()

        pltpu.sync_copy(acc_sh.at[sl], out_hbm.at[c, sl])

    return pl.kernel(body, out_type=out_type, mesh=_MESH,
                     scratch_types=scratch)


def _make_sc_deg():
    out_type = jax.ShapeDtypeStruct((NC, NP, 128), jnp.float32)
    scratch = [
        pltpu.VMEM((KID, CHUNK), jnp.int32),
        pltpu.VMEM((CHUNK, 128), jnp.float32),
        pltpu.VMEM_SHARED((NP, 128), jnp.float32),
    ]

    def body(dstw, z_hbm, ones_hbm, out_hbm, didx, ones_v, acc_sh):
        c = lax.axis_index("c")
        s = lax.axis_index("s")
        wid = c * NS + s
        sl = pl.ds(s * STRIPE, STRIPE)

        pltpu.sync_copy(z_hbm.at[sl], acc_sh.at[sl])
        pltpu.sync_copy(ones_hbm, ones_v)
        plsc.subcore_barrier()

        def blk(bi, carry):
            pltpu.sync_copy(dstw.at[wid, pl.ds(bi * KID, KID)], didx)

            def chunk(j, c2):
                pltpu.sync_copy(ones_v, acc_sh.at[didx.at[j]], add=True)
                return c2

            lax.fori_loop(0, KID, chunk, 0)
            return carry

        lax.fori_loop(0, K // KID, blk, 0)
        plsc.subcore_barrier()

        pltpu.sync_copy(acc_sh.at[sl], out_hbm.at[c, sl])

    return pl.kernel(body, out_type=out_type, mesh=_MESH,
                     scratch_types=scratch)


_AGG128 = _make_sc_agg()
_DEG = _make_sc_deg()

BR = 1024


def _tc0_body(x_ref, wn_ref, ws_ref, b_ref, m_ref, s_ref):
    xb = x_ref[...]
    m_ref[...] = jnp.dot(xb, wn_ref[...], preferred_element_type=jnp.float32)
    s_ref[...] = (jnp.dot(xb, ws_ref[...], preferred_element_type=jnp.float32)
                  + b_ref[...])


def _tc0(x, wn, ws, b):
    return pl.pallas_call(
        _tc0_body,
        grid=(NP // BR,),
        in_specs=[
            pl.BlockSpec((BR, 128), lambda r: (r, 0)),
            pl.BlockSpec((128, 128), lambda r: (0, 0)),
            pl.BlockSpec((128, 128), lambda r: (0, 0)),
            pl.BlockSpec((1, 128), lambda r: (0, 0)),
        ],
        out_specs=[pl.BlockSpec((BR, 128), lambda r: (r, 0)),
                   pl.BlockSpec((BR, 128), lambda r: (r, 0))],
        out_shape=[jax.ShapeDtypeStruct((NP, 128), jnp.float32)] * 2,
    )(x, wn, ws, b)


def _combine(s_ref, p_ref, degp_ref):
    deg = degp_ref[0, :, 0:1] + degp_ref[1, :, 0:1]
    invd = 1.0 / jnp.maximum(deg, 1.0)
    return s_ref[...] + (p_ref[0] + p_ref[1]) * invd


def _make_tc_mid(DN, DS):
    def tc_mid_body(s_ref, p_ref, degp_ref, wn_ref, ws_ref, b_ref,
                    m_ref, so_ref):
        h = jnp.maximum(_combine(s_ref, p_ref, degp_ref), 0.0)
        m_ref[...] = jnp.dot(h, wn_ref[...],
                             preferred_element_type=jnp.float32)
        so_ref[...] = (jnp.dot(h, ws_ref[...],
                               preferred_element_type=jnp.float32)
                       + b_ref[...])

    def tc_mid(s, p, degp, wn, ws, b):
        return pl.pallas_call(
            tc_mid_body,
            grid=(NP // BR,),
            in_specs=[
                pl.BlockSpec((BR, 128), lambda r: (r, 0)),
                pl.BlockSpec((NC, BR, 128), lambda r: (0, r, 0)),
                pl.BlockSpec((NC, BR, 128), lambda r: (0, r, 0)),
                pl.BlockSpec((128, DN), lambda r: (0, 0)),
                pl.BlockSpec((128, DS), lambda r: (0, 0)),
                pl.BlockSpec((1, DS), lambda r: (0, 0)),
            ],
            out_specs=[pl.BlockSpec((BR, DN), lambda r: (r, 0)),
                       pl.BlockSpec((BR, DS), lambda r: (r, 0))],
            out_shape=[jax.ShapeDtypeStruct((NP, DN), jnp.float32),
                       jax.ShapeDtypeStruct((NP, DS), jnp.float32)],
        )(s, p, degp, wn, ws, b)

    return tc_mid


_TC_MID128 = _make_tc_mid(128, 128)
_TC_MID2 = _make_tc_mid(128, 64)


def _tc_fin_body(s_ref, p_ref, degp_ref, o_ref):
    deg = degp_ref[0, :, 0:1] + degp_ref[1, :, 0:1]
    invd = 1.0 / jnp.maximum(deg, 1.0)
    o_ref[...] = s_ref[...] + (p_ref[0, :, :64] + p_ref[1, :, :64]) * invd


def _tc_fin(s, p, degp):
    return pl.pallas_call(
        _tc_fin_body,
        grid=(NP // BR,),
        in_specs=[
            pl.BlockSpec((BR, 64), lambda r: (r, 0)),
            pl.BlockSpec((NC, BR, 128), lambda r: (0, r, 0)),
            pl.BlockSpec((NC, BR, 128), lambda r: (0, r, 0)),
        ],
        out_specs=pl.BlockSpec((BR, 64), lambda r: (r, 0)),
        out_shape=jax.ShapeDtypeStruct((NP, 64), jnp.float32),
    )(s, p, degp)


@jax.jit
def kernel(x, edge_index, W_self0, W_neigh0, b0, W_self1, W_neigh1, b1,
           W_self2, W_neigh2, b2):
    xp = jnp.pad(x, ((0, NP - N), (0, 0)))
    src = jnp.pad(edge_index[0], (0, EP - E)).reshape(NW, K, CHUNK)
    dst = jnp.pad(edge_index[1], (0, EP - E),
                  constant_values=N).reshape(NW, K, CHUNK)
    z128 = jnp.zeros((NP, 128), jnp.float32)
    ones = jnp.ones((CHUNK, 128), jnp.float32)
    wn2p = jnp.pad(W_neigh2, ((0, 0), (0, 128 - 64)))

    degp = _DEG(dst, z128, ones)
    m0, s0 = _tc0(xp, W_neigh0, W_self0, b0.reshape(1, -1))
    p0 = _AGG128(m0, src, dst, z128)
    m1, s1 = _TC_MID128(s0, p0, degp, W_neigh1, W_self1, b1.reshape(1, -1))
    p1 = _AGG128(m1, src, dst, z128)
    m2, s2 = _TC_MID2(s1, p1, degp, wn2p, W_self2, b2.reshape(1, -1))
    p2 = _AGG128(m2, src, dst, z128)
    out = _tc_fin(s2, p2, degp)
    return out[:N]
